# Initial kernel scaffold; baseline (speedup 1.0000x reference)
import functools

import jax
import jax.numpy as jnp
from jax import lax
from jax.experimental import pallas as pl
from jax.experimental.pallas import tpu as pltpu
from jax.experimental.pallas import tpu_sc as plsc

_NCORES = 2
_NSUB = 16
_NW = _NCORES * _NSUB
_L = 16

_NB = 5
_NCF = 5
_NK = 16
_D = 64
_TOK = 1 + _NB + _NCF + _NK
_CB = 16


@functools.partial(jax.jit, static_argnames=("batch", "vocab"))
def _tokenize(batch, vocab, bin_idx_f, cat_idx_f, vals_f, bin_tbl, w_tbl,
              b_tbl, m_tbl, cls_v, cat_flat):
    rows_per_w = batch // _NW
    nchunks = rows_per_w // _CB
    dq = _D // _L

    mesh = plsc.VectorSubcoreMesh(
        core_axis_name="c", subcore_axis_name="s",
        num_cores=_NCORES, num_subcores=_NSUB)

    @functools.partial(
        pl.kernel,
        out_type=jax.ShapeDtypeStruct((batch * _TOK, _D), jnp.float32),
        mesh=mesh,
        scratch_types=[
            pltpu.VMEM((_CB * _NB,), jnp.int32),
            pltpu.VMEM((_CB * _NK,), jnp.int32),
            pltpu.VMEM((_CB * _NCF,), jnp.float32),
            pltpu.VMEM((_CB * _NK,), jnp.int32),
            pltpu.VMEM((_NB * 2 * _D,), jnp.float32),
            pltpu.VMEM((_NCF * _D,), jnp.float32),
            pltpu.VMEM((_NCF * _D,), jnp.float32),
            pltpu.VMEM((_NCF * _D,), jnp.float32),
            pltpu.VMEM((_D,), jnp.float32),
            pltpu.VMEM((_CB * _TOK, _D), jnp.float32),
            pltpu.SemaphoreType.DMA,
        ],
    )
    def tok_kernel(binidx_hbm, catidx_hbm, vals_hbm, bintbl_hbm, wtbl_hbm,
                   btbl_hbm, mtbl_hbm, cls_hbm, cat_hbm, out_hbm,
                   bidx_v, cidx_v, vals_v, gidx_v, btbl_v, wv, bv, mv,
                   clsv, stage, sem):
        wid = lax.axis_index("s") * _NCORES + lax.axis_index("c")
        base0 = wid * rows_per_w

        pltpu.sync_copy(bintbl_hbm, btbl_v)
        pltpu.sync_copy(wtbl_hbm, wv)
        pltpu.sync_copy(btbl_hbm, bv)
        pltpu.sync_copy(mtbl_hbm, mv)
        pltpu.sync_copy(cls_hbm, clsv)

        viota = lax.iota(jnp.int32, _L) * vocab

        def chunk(c, carry):
            b0 = base0 + c * _CB
            pltpu.sync_copy(binidx_hbm.at[pl.ds(b0 * _NB, _CB * _NB)], bidx_v)
            pltpu.sync_copy(vals_hbm.at[pl.ds(b0 * _NCF, _CB * _NCF)], vals_v)
            pltpu.sync_copy(catidx_hbm.at[pl.ds(b0 * _NK, _CB * _NK)], cidx_v)

            for bl in range(_CB):
                gidx_v[pl.ds(bl * _NK, _NK)] = (
                    cidx_v[pl.ds(bl * _NK, _NK)] + viota)
            copies = []
            for bl in range(_CB):
                cp = pltpu.make_async_copy(
                    cat_hbm.at[gidx_v.at[pl.ds(bl * _NK, _NK)]],
                    stage.at[pl.ds(bl * _TOK + 1 + _NB + _NCF, _NK), :],
                    sem)
                cp.start()
                copies.append(cp)

            for q in range(dq):
                cvec = clsv[pl.ds(q * _L, _L)]
                for bl in range(_CB):
                    stage[bl * _TOK, pl.ds(q * _L, _L)] = cvec

            for f in range(_NB):
                msks = []
                for bl in range(_CB):
                    xb = plsc.load_gather(
                        bidx_v, [jnp.full((_L,), bl * _NB + f, jnp.int32)])
                    msks.append(xb == 1)
                for q in range(dq):
                    e0 = btbl_v[pl.ds(f * 2 * _D + q * _L, _L)]
                    e1 = btbl_v[pl.ds(f * 2 * _D + _D + q * _L, _L)]
                    for bl in range(_CB):
                        stage[bl * _TOK + 1 + f, pl.ds(q * _L, _L)] = (
                            jnp.where(msks[bl], e1, e0))

            for f in range(_NCF):
                misses, vzs = [], []
                for bl in range(_CB):
                    v = plsc.load_gather(
                        vals_v, [jnp.full((_L,), bl * _NCF + f, jnp.int32)])
                    miss = v != v
                    misses.append(miss)
                    vzs.append(jnp.where(miss, 0.0, v))
                for q in range(dq):
                    wq = wv[pl.ds(f * _D + q * _L, _L)]
                    bq = bv[pl.ds(f * _D + q * _L, _L)]
                    mq = mv[pl.ds(f * _D + q * _L, _L)]
                    for bl in range(_CB):
                        stage[bl * _TOK + 1 + _NB + f, pl.ds(q * _L, _L)] = (
                            jnp.where(misses[bl], mq, vzs[bl] * wq + bq))

            for cp in copies:
                cp.wait()
            pltpu.sync_copy(stage,
                            out_hbm.at[pl.ds(b0 * _TOK, _CB * _TOK), :])
            return carry

        lax.fori_loop(0, nchunks, chunk, 0)

    return tok_kernel(bin_idx_f, cat_idx_f, vals_f, bin_tbl, w_tbl, b_tbl,
                      m_tbl, cls_v, cat_flat)


def kernel(bin_idx, cat_idx, cont_vals, bin_emb, cont_w, cont_b, cont_mask,
           cat_emb, cls):
    batch = bin_idx.shape[0]
    vocab = cat_emb.shape[1]
    out = _tokenize(
        batch, vocab,
        bin_idx.reshape(-1), cat_idx.reshape(-1), cont_vals.reshape(-1),
        bin_emb.reshape(-1), cont_w.reshape(-1), cont_b.reshape(-1),
        cont_mask.reshape(-1), cls,
        cat_emb.reshape(-1, cat_emb.shape[2]))
    return out.reshape(batch, _TOK, _D)


# speedup vs baseline: 1.0043x; 1.0043x over previous
import functools

import jax
import jax.numpy as jnp
from jax import lax
from jax.experimental import pallas as pl
from jax.experimental.pallas import tpu as pltpu
from jax.experimental.pallas import tpu_sc as plsc

_NCORES = 2
_NSUB = 16
_NW = _NCORES * _NSUB
_L = 16

_NB = 5
_NCF = 5
_NK = 16
_D = 64
_TOK = 1 + _NB + _NCF + _NK
_CB = 16


@functools.partial(jax.jit, static_argnames=("batch", "vocab"))
def _tokenize(batch, vocab, bin_idx_f, cat_idx_f, vals_f, bin_tbl, w_tbl,
              b_tbl, m_tbl, cls_v, cat_flat):
    rows_per_w = batch // _NW
    nchunks = rows_per_w // _CB
    dq = _D // _L

    mesh = plsc.VectorSubcoreMesh(
        core_axis_name="c", subcore_axis_name="s",
        num_cores=_NCORES, num_subcores=_NSUB)

    @functools.partial(
        pl.kernel,
        out_type=jax.ShapeDtypeStruct((batch * _TOK, _D), jnp.float32),
        mesh=mesh,
        scratch_types=[
            pltpu.VMEM((_CB * _NB * _L,), jnp.int32),
            pltpu.VMEM((_CB * _NK,), jnp.int32),
            pltpu.VMEM((_CB * _NCF * _L,), jnp.float32),
            pltpu.VMEM((_CB * _NK,), jnp.int32),
            pltpu.VMEM((_NB * 2 * _D,), jnp.float32),
            pltpu.VMEM((_NCF * _D,), jnp.float32),
            pltpu.VMEM((_NCF * _D,), jnp.float32),
            pltpu.VMEM((_NCF * _D,), jnp.float32),
            pltpu.VMEM((_NB * _D,), jnp.float32),
            pltpu.VMEM((_D,), jnp.float32),
            pltpu.VMEM((_CB * _TOK, _D), jnp.float32),
            pltpu.SemaphoreType.DMA,
        ],
        compiler_params=pltpu.CompilerParams(use_tc_tiling_on_sc=False),
    )
    def tok_kernel(binidx_hbm, catidx_hbm, vals_hbm, bintbl_hbm, wtbl_hbm,
                   btbl_hbm, mtbl_hbm, cls_hbm, cat_hbm, out_hbm,
                   bidx_v, cidx_v, vals_v, gidx_v, btbl_v, wv, bv, mv,
                   diff_v, clsv, stage, sem):
        wid = lax.axis_index("s") * _NCORES + lax.axis_index("c")
        base0 = wid * rows_per_w

        pltpu.sync_copy(bintbl_hbm, btbl_v)
        pltpu.sync_copy(wtbl_hbm, wv)
        pltpu.sync_copy(btbl_hbm, bv)
        pltpu.sync_copy(mtbl_hbm, mv)
        pltpu.sync_copy(cls_hbm, clsv)

        for f in range(_NB):
            for q in range(_D // _L):
                diff_v[pl.ds(f * _D + q * _L, _L)] = (
                    btbl_v[pl.ds(f * 2 * _D + _D + q * _L, _L)]
                    - btbl_v[pl.ds(f * 2 * _D + q * _L, _L)])

        viota = lax.iota(jnp.int32, _L) * vocab

        def chunk(c, carry):
            b0 = base0 + c * _CB
            pltpu.sync_copy(binidx_hbm.at[pl.ds(b0 * _NB * _L,
                                                _CB * _NB * _L)], bidx_v)
            pltpu.sync_copy(vals_hbm.at[pl.ds(b0 * _NCF * _L,
                                              _CB * _NCF * _L)], vals_v)
            pltpu.sync_copy(catidx_hbm.at[pl.ds(b0 * _NK, _CB * _NK)], cidx_v)

            for bl in range(_CB):
                gidx_v[pl.ds(bl * _NK, _NK)] = (
                    cidx_v[pl.ds(bl * _NK, _NK)] + viota)
            copies = []
            for bl in range(_CB):
                cp = pltpu.make_async_copy(
                    cat_hbm.at[gidx_v.at[pl.ds(bl * _NK, _NK)]],
                    stage.at[pl.ds(bl * _TOK + 1 + _NB + _NCF, _NK), :],
                    sem)
                cp.start()
                copies.append(cp)

            for q in range(dq):
                cvec = clsv[pl.ds(q * _L, _L)]
                for bl in range(_CB):
                    stage[bl * _TOK, pl.ds(q * _L, _L)] = cvec

            for f in range(_NB):
                e0s = [btbl_v[pl.ds(f * 2 * _D + q * _L, _L)]
                       for q in range(dq)]
                dfs = [diff_v[pl.ds(f * _D + q * _L, _L)] for q in range(dq)]
                for bl in range(_CB):
                    xf = bidx_v[pl.ds((bl * _NB + f) * _L, _L)].astype(
                        jnp.float32)
                    for q in range(dq):
                        stage[bl * _TOK + 1 + f, pl.ds(q * _L, _L)] = (
                            e0s[q] + xf * dfs[q])

            for f in range(_NCF):
                wqs = [wv[pl.ds(f * _D + q * _L, _L)] for q in range(dq)]
                bqs = [bv[pl.ds(f * _D + q * _L, _L)] for q in range(dq)]
                mqs = [mv[pl.ds(f * _D + q * _L, _L)] for q in range(dq)]
                for bl in range(_CB):
                    vs = vals_v[pl.ds((bl * _NCF + f) * _L, _L)]
                    miss = vs != vs
                    vz = jnp.where(miss, jnp.float32(0), vs)
                    for q in range(dq):
                        stage[bl * _TOK + 1 + _NB + f, pl.ds(q * _L, _L)] = (
                            jnp.where(miss, mqs[q], vz * wqs[q] + bqs[q]))

            for cp in copies:
                cp.wait()
            pltpu.sync_copy(stage,
                            out_hbm.at[pl.ds(b0 * _TOK, _CB * _TOK), :])
            return carry

        lax.fori_loop(0, nchunks, chunk, 0)

    return tok_kernel(bin_idx_f, cat_idx_f, vals_f, bin_tbl, w_tbl, b_tbl,
                      m_tbl, cls_v, cat_flat)


def kernel(bin_idx, cat_idx, cont_vals, bin_emb, cont_w, cont_b, cont_mask,
           cat_emb, cls):
    batch = bin_idx.shape[0]
    vocab = cat_emb.shape[1]
    bin_rep = jnp.broadcast_to(
        bin_idx.reshape(-1, 1), (batch * _NB, _L)).reshape(-1)
    val_rep = jnp.broadcast_to(
        cont_vals.reshape(-1, 1), (batch * _NCF, _L)).reshape(-1)
    out = _tokenize(
        batch, vocab,
        bin_rep, cat_idx.reshape(-1), val_rep,
        bin_emb.reshape(-1), cont_w.reshape(-1), cont_b.reshape(-1),
        cont_mask.reshape(-1), cls,
        cat_emb.reshape(-1, cat_emb.shape[2]))
    return out.reshape(batch, _TOK, _D)


# speedup vs baseline: 2.3343x; 2.3245x over previous
import functools

import jax
import jax.numpy as jnp
from jax import lax
from jax.experimental import pallas as pl
from jax.experimental.pallas import tpu as pltpu
from jax.experimental.pallas import tpu_sc as plsc

_NCORES = 2
_NSUB = 16
_NW = _NCORES * _NSUB
_L = 16

_NB = 5
_NCF = 5
_NK = 16
_D = 64
_TOK = 1 + _NB + _NCF + _NK


@functools.partial(jax.jit, static_argnames=("batch", "vocab"))
def _tokenize(batch, vocab, bidx_t, cidx_t, cval_ti, bin_tbl, w_tbl,
              b_tbl, m_tbl, cls_v, cat_t):
    half = batch // 2
    grp = half // _L
    dense_rows = (1 + _NB + _NCF) * _D
    cls_per_w = _D // _NW
    bc_per_w = (_NB * _D) // _NW
    d_half = _D // 2

    mesh = plsc.VectorSubcoreMesh(
        core_axis_name="c", subcore_axis_name="s",
        num_cores=_NCORES, num_subcores=_NSUB)

    @functools.partial(
        pl.kernel,
        out_type=jax.ShapeDtypeStruct((_TOK, _D, batch), jnp.float32),
        mesh=mesh,
        scratch_types=[
            pltpu.VMEM((vocab,), jnp.float32),
            pltpu.VMEM((batch,), jnp.int32),
            pltpu.VMEM((half,), jnp.float32),
            pltpu.VMEM((_NB * 2 * _D,), jnp.float32),
            pltpu.VMEM((_NCF * _D,), jnp.float32),
            pltpu.VMEM((_NCF * _D,), jnp.float32),
            pltpu.VMEM((_NCF * _D,), jnp.float32),
            pltpu.VMEM((_D,), jnp.float32),
        ],
        compiler_params=pltpu.CompilerParams(
            use_tc_tiling_on_sc=True, needs_layout_passes=False),
    )
    def tok_kernel(bidx_h, cidx_h, cval_h, bintbl_h, wtbl_h, btbl_h,
                   mtbl_h, cls_h, cat_h, out_h,
                   tblrow, colbuf, outbuf, binv, wv, bv, mv, clsv):
        w = lax.axis_index("s") * _NCORES + lax.axis_index("c")

        pltpu.sync_copy(bintbl_h, binv)
        pltpu.sync_copy(wtbl_h, wv)
        pltpu.sync_copy(btbl_h, bv)
        pltpu.sync_copy(mtbl_h, mv)
        pltpu.sync_copy(cls_h, clsv)

        def splat(tbl, pos):
            return plsc.load_gather(tbl, [jnp.full((_L,), pos, jnp.int32)])

        for i in range(cls_per_w):
            d = w * cls_per_w + i
            s = splat(clsv, d)
            for h in range(2):
                def cls_body(j, _):
                    outbuf[pl.ds(j * _L, _L)] = s
                    return 0
                lax.fori_loop(0, grp, cls_body, 0)
                pltpu.sync_copy(outbuf, out_h.at[0, d, pl.ds(h * half, half)])

        for i in range(bc_per_w):
            r = w * bc_per_w + i
            f = r // _D
            d = r % _D
            pltpu.sync_copy(bidx_h.at[f, :], colbuf)
            e0 = splat(binv, f * 2 * _D + d)
            e1 = splat(binv, f * 2 * _D + _D + d)
            df = e1 - e0
            for h in range(2):
                def bin_body(j, _):
                    x = colbuf[pl.ds(h * half + j * _L, _L)]
                    outbuf[pl.ds(j * _L, _L)] = e0 + x.astype(jnp.float32) * df
                    return 0
                lax.fori_loop(0, grp, bin_body, 0)
                pltpu.sync_copy(outbuf,
                                out_h.at[1 + f, d, pl.ds(h * half, half)])

        for i in range(bc_per_w):
            r = w * bc_per_w + i
            f = r // _D
            d = r % _D
            pltpu.sync_copy(cval_h.at[f, :], colbuf)
            ws = splat(wv, f * _D + d)
            bs = splat(bv, f * _D + d)
            ms = splat(mv, f * _D + d)
            for h in range(2):
                def cont_body(j, _):
                    xi = colbuf[pl.ds(h * half + j * _L, _L)]
                    v = plsc.bitcast(xi, jnp.float32)
                    miss = v != v
                    vz = jnp.where(miss, jnp.float32(0), v)
                    outbuf[pl.ds(j * _L, _L)] = jnp.where(
                        miss, ms, vz * ws + bs)
                    return 0
                lax.fori_loop(0, grp, cont_body, 0)
                pltpu.sync_copy(outbuf,
                                out_h.at[1 + _NB + f, d, pl.ds(h * half, half)])

        f = w // 2
        d0 = (w % 2) * d_half
        pltpu.sync_copy(cidx_h.at[f, :], colbuf)

        def cat_row(dd, _):
            d = d0 + dd
            pltpu.sync_copy(cat_h.at[f, d, :], tblrow)
            for h in range(2):
                def cat_body(j, _):
                    idx = colbuf[pl.ds(h * half + j * _L, _L)]
                    outbuf[pl.ds(j * _L, _L)] = plsc.load_gather(tblrow, [idx])
                    return 0
                lax.fori_loop(0, grp, cat_body, 0)
                pltpu.sync_copy(
                    outbuf,
                    out_h.at[1 + _NB + _NCF + f, d, pl.ds(h * half, half)])
            return 0

        lax.fori_loop(0, d_half, cat_row, 0)

    return tok_kernel(bidx_t, cidx_t, cval_ti, bin_tbl, w_tbl, b_tbl,
                      m_tbl, cls_v, cat_t)


def kernel(bin_idx, cat_idx, cont_vals, bin_emb, cont_w, cont_b, cont_mask,
           cat_emb, cls):
    batch = bin_idx.shape[0]
    vocab = cat_emb.shape[1]
    cat_t = cat_emb.transpose(0, 2, 1)
    cidx_t = cat_idx.T
    bidx_t = bin_idx.T
    cval_ti = lax.bitcast_convert_type(cont_vals.T, jnp.int32)
    out = _tokenize(
        batch, vocab, bidx_t, cidx_t, cval_ti,
        bin_emb.reshape(-1), cont_w.reshape(-1), cont_b.reshape(-1),
        cont_mask.reshape(-1), cls, cat_t)
    return out.transpose(2, 0, 1)


# speedup vs baseline: 3.5366x; 1.5150x over previous
import functools

import jax
import jax.numpy as jnp
from jax import lax
from jax.experimental import pallas as pl
from jax.experimental.pallas import tpu as pltpu
from jax.experimental.pallas import tpu_sc as plsc

_NCORES = 2
_NSUB = 16
_NW = _NCORES * _NSUB
_L = 16

_NB = 5
_NCF = 5
_NK = 16
_D = 64
_TOK = 1 + _NB + _NCF + _NK


@functools.partial(jax.jit, static_argnames=("batch", "vocab"))
def _tokenize(batch, vocab, bidx_t, cidx_t, cval_ti, bin_tbl, w_tbl,
              b_tbl, m_tbl, cls_v, cat_t):
    half = batch // 2
    grp = half // _L
    unroll = 8
    grp_u = grp // unroll
    dense_rows = (1 + _NB + _NCF) * _D
    cls_per_w = _D // _NW
    bc_per_w = (_NB * _D) // _NW
    d_half = _D // 2

    mesh = plsc.VectorSubcoreMesh(
        core_axis_name="c", subcore_axis_name="s",
        num_cores=_NCORES, num_subcores=_NSUB)

    @functools.partial(
        pl.kernel,
        out_type=jax.ShapeDtypeStruct((_TOK, _D, batch), jnp.float32),
        mesh=mesh,
        scratch_types=[
            pltpu.VMEM((vocab,), jnp.float32),
            pltpu.VMEM((batch,), jnp.int32),
            pltpu.VMEM((half,), jnp.float32),
            pltpu.VMEM((_NB * 2 * _D,), jnp.float32),
            pltpu.VMEM((_NCF * _D,), jnp.float32),
            pltpu.VMEM((_NCF * _D,), jnp.float32),
            pltpu.VMEM((_NCF * _D,), jnp.float32),
            pltpu.VMEM((_D,), jnp.float32),
        ],
        compiler_params=pltpu.CompilerParams(
            use_tc_tiling_on_sc=True, needs_layout_passes=False),
    )
    def tok_kernel(bidx_h, cidx_h, cval_h, bintbl_h, wtbl_h, btbl_h,
                   mtbl_h, cls_h, cat_h, out_h,
                   tblrow, colbuf, outbuf, binv, wv, bv, mv, clsv):
        w = lax.axis_index("s") * _NCORES + lax.axis_index("c")

        pltpu.sync_copy(bintbl_h, binv)
        pltpu.sync_copy(wtbl_h, wv)
        pltpu.sync_copy(btbl_h, bv)
        pltpu.sync_copy(mtbl_h, mv)
        pltpu.sync_copy(cls_h, clsv)

        def splat(tbl, pos):
            return plsc.load_gather(tbl, [jnp.full((_L,), pos, jnp.int32)])

        for i in range(cls_per_w):
            d = w * cls_per_w + i
            s = splat(clsv, d)
            for h in range(2):
                def cls_body(j, _):
                    for u in range(unroll):
                        outbuf[pl.ds((j * unroll + u) * _L, _L)] = s
                    return 0
                lax.fori_loop(0, grp_u, cls_body, 0)
                pltpu.sync_copy(outbuf, out_h.at[0, d, pl.ds(h * half, half)])

        for i in range(bc_per_w):
            r = w * bc_per_w + i
            f = r // _D
            d = r % _D
            pltpu.sync_copy(bidx_h.at[f, :], colbuf)
            e0 = splat(binv, f * 2 * _D + d)
            e1 = splat(binv, f * 2 * _D + _D + d)
            df = e1 - e0
            for h in range(2):
                def bin_body(j, _):
                    for u in range(unroll):
                        g = j * unroll + u
                        x = colbuf[pl.ds(h * half + g * _L, _L)]
                        outbuf[pl.ds(g * _L, _L)] = (
                            e0 + x.astype(jnp.float32) * df)
                    return 0
                lax.fori_loop(0, grp_u, bin_body, 0)
                pltpu.sync_copy(outbuf,
                                out_h.at[1 + f, d, pl.ds(h * half, half)])

        for i in range(bc_per_w):
            r = w * bc_per_w + i
            f = r // _D
            d = r % _D
            pltpu.sync_copy(cval_h.at[f, :], colbuf)
            ws = splat(wv, f * _D + d)
            bs = splat(bv, f * _D + d)
            ms = splat(mv, f * _D + d)
            for h in range(2):
                def cont_body(j, _):
                    for u in range(unroll):
                        g = j * unroll + u
                        xi = colbuf[pl.ds(h * half + g * _L, _L)]
                        v = plsc.bitcast(xi, jnp.float32)
                        miss = v != v
                        vz = jnp.where(miss, jnp.float32(0), v)
                        outbuf[pl.ds(g * _L, _L)] = jnp.where(
                            miss, ms, vz * ws + bs)
                    return 0
                lax.fori_loop(0, grp_u, cont_body, 0)
                pltpu.sync_copy(outbuf,
                                out_h.at[1 + _NB + f, d, pl.ds(h * half, half)])

        f = w // 2
        d0 = (w % 2) * d_half
        pltpu.sync_copy(cidx_h.at[f, :], colbuf)

        def cat_row(dd, _):
            d = d0 + dd
            pltpu.sync_copy(cat_h.at[f, d, :], tblrow)
            for h in range(2):
                def cat_body(j, _):
                    for u in range(unroll):
                        g = j * unroll + u
                        idx = colbuf[pl.ds(h * half + g * _L, _L)]
                        outbuf[pl.ds(g * _L, _L)] = plsc.load_gather(
                            tblrow, [idx])
                    return 0
                lax.fori_loop(0, grp_u, cat_body, 0)
                pltpu.sync_copy(
                    outbuf,
                    out_h.at[1 + _NB + _NCF + f, d, pl.ds(h * half, half)])
            return 0

        lax.fori_loop(0, d_half, cat_row, 0)

    return tok_kernel(bidx_t, cidx_t, cval_ti, bin_tbl, w_tbl, b_tbl,
                      m_tbl, cls_v, cat_t)


def kernel(bin_idx, cat_idx, cont_vals, bin_emb, cont_w, cont_b, cont_mask,
           cat_emb, cls):
    batch = bin_idx.shape[0]
    vocab = cat_emb.shape[1]
    cat_t = cat_emb.transpose(0, 2, 1)
    cidx_t = cat_idx.T
    bidx_t = bin_idx.T
    cval_ti = lax.bitcast_convert_type(cont_vals.T, jnp.int32)
    out = _tokenize(
        batch, vocab, bidx_t, cidx_t, cval_ti,
        bin_emb.reshape(-1), cont_w.reshape(-1), cont_b.reshape(-1),
        cont_mask.reshape(-1), cls, cat_t)
    return out.transpose(2, 0, 1)
